# baseline (speedup 1.0000x reference)
import functools

import jax
import jax.numpy as jnp
from jax import lax
from jax.experimental import pallas as pl
from jax.experimental.pallas import tpu as pltpu
from jax.experimental.pallas import tpu_sc as plsc

NE = 16
ES = 384
D = 768
T = 2048
TILE = 128
PAD = 4096
GR = PAD // TILE
GS = T // TILE
G = GR + GS
NSLOT = PAD + T


def _cumsum0(a):
    n = a.shape[0]
    sh = 1
    while sh < n:
        z = jnp.zeros((sh,) + a.shape[1:], dtype=a.dtype)
        a = a + jnp.concatenate([z, a[:-sh]], axis=0)
        sh *= 2
    return a


def _cumsum1(a):
    n = a.shape[1]
    sh = 1
    while sh < n:
        z = jnp.zeros(a.shape[:1] + (sh,), dtype=a.dtype)
        a = a + jnp.concatenate([z, a[:, :-sh]], axis=1)
        sh *= 2
    return a


def _router_body(x_ref, wr_ref, br_ref, bias_ref,
                 dest_ref, order_ref, gates_ref, texp_ref):
    xl = x_ref[...]
    logits = jnp.dot(xl, wr_ref[...], preferred_element_type=jnp.float32)
    logits = logits + br_ref[...] + bias_ref[...]
    lmax = jnp.max(logits, axis=1, keepdims=True)
    gate = 1.0 / jnp.sum(jnp.exp(logits - lmax), axis=1, keepdims=True)
    lane = lax.broadcasted_iota(jnp.int32, (T, NE), 1)
    eid = jnp.min(jnp.where(logits == lmax, lane, NE), axis=1, keepdims=True)
    onehot = (lane == eid).astype(jnp.float32)
    counts = jnp.sum(onehot, axis=0, keepdims=True).astype(jnp.int32)
    ptrows = ((counts + TILE - 1) // TILE) * TILE
    poff = _cumsum1(ptrows) - ptrows
    cum = _cumsum0(onehot) - onehot
    rank = jnp.sum(cum * onehot, axis=1, keepdims=True)
    destf = jnp.sum(onehot * poff.astype(jnp.float32), axis=1,
                    keepdims=True) + rank
    dest = destf.astype(jnp.int32)
    dest_ref[...] = dest

    trow = lax.broadcasted_iota(jnp.int32, (G, NE), 0)
    te = jnp.sum((trow * TILE >= poff).astype(jnp.int32), axis=1,
                 keepdims=True) - 1
    gi = lax.broadcasted_iota(jnp.int32, (G, 1), 0)
    texp_ref[...] = jnp.where(gi >= GR, NE, te)

    tok = lax.broadcasted_iota(jnp.int32, (T, 1), 0).astype(jnp.float32)
    for r in range(PAD // 256):
        slots = r * 256 + lax.broadcasted_iota(jnp.int32, (1, 256), 1)
        m = (dest == slots).astype(jnp.float32)
        occ = jnp.sum(m, axis=0, keepdims=True)
        osum = jnp.sum(m * tok, axis=0, keepdims=True).astype(jnp.int32)
        order_ref[r:r + 1, :] = jnp.where(occ > 0.0, osum,
                                          jnp.bitwise_and(slots, T - 1))
        gates_ref[r:r + 1, :] = jnp.sum(m * gate, axis=0, keepdims=True)


def _gelu(h):
    return 0.5 * h * (1.0 + lax.erf(h * 0.7071067811865476))


def _ffn_body(texp_ref, xs_ref, w1_ref, b1_ref, w2_ref, b2_ref,
              g_ref, ys_ref):
    xb = xs_ref[...].astype(jnp.bfloat16)
    h = jnp.dot(xb, w1_ref[0], preferred_element_type=jnp.float32)
    h = _gelu(h + b1_ref[0]).astype(jnp.bfloat16)
    y = jnp.dot(h, w2_ref[0], preferred_element_type=jnp.float32)
    ys_ref[...] = (y + b2_ref[0]) * g_ref[...]


def _shared_body(x_ref, w1_ref, b1_ref, w2_ref, b2_ref, ys_ref):
    xb = x_ref[...].astype(jnp.bfloat16)
    h = jnp.dot(xb, w1_ref[...], preferred_element_type=jnp.float32)
    h = _gelu(h + b1_ref[...]).astype(jnp.bfloat16)
    y = jnp.dot(h, w2_ref[...], preferred_element_type=jnp.float32)
    ys_ref[...] = y + b2_ref[...]


def _add_body(a_ref, b_ref, o_ref):
    o_ref[...] = a_ref[...] + b_ref[...]


def _dispatch_body(x_hbm, order_hbm, xs_hbm, idx_v, rows_v, sem):
    wid = lax.axis_index("s") * 2 + lax.axis_index("c")
    base = wid * (PAD // 32)
    pltpu.sync_copy(order_hbm.at[pl.ds(base, PAD // 32)], idx_v)
    pltpu.async_copy(x_hbm.at[idx_v], rows_v, sem).wait()
    pltpu.sync_copy(rows_v, xs_hbm.at[pl.ds(base, PAD // 32)])


def _combine_body(ys_hbm, dest_hbm, out_hbm, idx_v, rows_v, sem):
    wid = lax.axis_index("s") * 2 + lax.axis_index("c")
    n = T // 32
    base = wid * n
    pltpu.sync_copy(dest_hbm.at[pl.ds(base, n)], idx_v)
    pltpu.async_copy(ys_hbm.at[idx_v], rows_v, sem).wait()
    pltpu.sync_copy(rows_v, out_hbm.at[pl.ds(base, n)])


def _make_router(interpret=False):
    return pl.pallas_call(
        _router_body,
        out_shape=(
            jax.ShapeDtypeStruct((T, 1), jnp.int32),
            jax.ShapeDtypeStruct((PAD // 256, 256), jnp.int32),
            jax.ShapeDtypeStruct((PAD // 256, 256), jnp.float32),
            jax.ShapeDtypeStruct((G, 1), jnp.int32),
        ),
        interpret=interpret,
    )


def _make_ffn(interpret=False):
    grid_spec = pltpu.PrefetchScalarGridSpec(
        num_scalar_prefetch=1,
        grid=(GR,),
        in_specs=[
            pl.BlockSpec((TILE, D), lambda i, te: (i, 0)),
            pl.BlockSpec((1, D, ES), lambda i, te: (te[i], 0, 0)),
            pl.BlockSpec((1, 1, ES), lambda i, te: (te[i], 0, 0)),
            pl.BlockSpec((1, ES, D), lambda i, te: (te[i], 0, 0)),
            pl.BlockSpec((1, 1, D), lambda i, te: (te[i], 0, 0)),
            pl.BlockSpec((TILE, 1), lambda i, te: (i, 0)),
        ],
        out_specs=pl.BlockSpec((TILE, D), lambda i, te: (i, 0)),
    )
    return pl.pallas_call(
        _ffn_body,
        grid_spec=grid_spec,
        out_shape=jax.ShapeDtypeStruct((PAD, D), jnp.float32),
        compiler_params=pltpu.CompilerParams(
            dimension_semantics=("arbitrary",)),
        interpret=interpret,
    )


def _make_shared(interpret=False):
    return pl.pallas_call(
        _shared_body,
        grid=(GS,),
        in_specs=[
            pl.BlockSpec((TILE, D), lambda i: (i, 0)),
            pl.BlockSpec((D, ES), lambda i: (0, 0)),
            pl.BlockSpec((1, ES), lambda i: (0, 0)),
            pl.BlockSpec((ES, D), lambda i: (0, 0)),
            pl.BlockSpec((1, D), lambda i: (0, 0)),
        ],
        out_specs=pl.BlockSpec((TILE, D), lambda i: (i, 0)),
        out_shape=jax.ShapeDtypeStruct((T, D), jnp.float32),
        compiler_params=pltpu.CompilerParams(
            dimension_semantics=("arbitrary",)),
        interpret=interpret,
    )


def _make_add(interpret=False):
    return pl.pallas_call(
        _add_body,
        grid=(8,),
        in_specs=[
            pl.BlockSpec((T // 8, D), lambda i: (i, 0)),
            pl.BlockSpec((T // 8, D), lambda i: (i, 0)),
        ],
        out_specs=pl.BlockSpec((T // 8, D), lambda i: (i, 0)),
        out_shape=jax.ShapeDtypeStruct((T, D), jnp.float32),
        interpret=interpret,
    )


def _make_dispatch():
    mesh = plsc.VectorSubcoreMesh(core_axis_name="c", subcore_axis_name="s")
    return pl.kernel(
        _dispatch_body,
        out_type=jax.ShapeDtypeStruct((PAD, D), jnp.float32),
        mesh=mesh,
        scratch_types=[
            pltpu.VMEM((PAD // 32,), jnp.int32),
            pltpu.VMEM((PAD // 32, D), jnp.float32),
            pltpu.SemaphoreType.DMA,
        ],
    )


def _make_combine():
    mesh = plsc.VectorSubcoreMesh(core_axis_name="c", subcore_axis_name="s")
    return pl.kernel(
        _combine_body,
        out_type=jax.ShapeDtypeStruct((T, D), jnp.float32),
        mesh=mesh,
        scratch_types=[
            pltpu.VMEM((T // 32,), jnp.int32),
            pltpu.VMEM((T // 32, D), jnp.float32),
            pltpu.SemaphoreType.DMA,
        ],
    )


@jax.jit
def kernel(x, Wr, br, bias, W1, b1, W2, b2):
    x2 = x.reshape(T, D)
    dest2d, order2d, gates2d, texp2d = _make_router()(
        x2, Wr, br.reshape(1, NE), bias.reshape(1, NE))
    order = order2d.reshape(PAD)
    dest = dest2d.reshape(T)
    gates = gates2d.reshape(PAD, 1)
    texp = texp2d.reshape(G)
    W1b = W1.astype(jnp.bfloat16)
    W2b = W2.astype(jnp.bfloat16)
    xs = _make_dispatch()(x2, order)
    ys_sh = _make_shared()(x2, W1b[NE], b1[NE].reshape(1, ES),
                           W2b[NE], b2[NE].reshape(1, D))
    ys = _make_ffn()(texp, xs, W1b, b1.reshape(NE + 1, 1, ES), W2b,
                     b2.reshape(NE + 1, 1, D), gates)
    ys_r = _make_combine()(ys, dest)
    out = _make_add()(ys_r, ys_sh)
    return out.reshape(x.shape)


# speedup vs baseline: 1.1502x; 1.1502x over previous
import functools

import jax
import jax.numpy as jnp
from jax import lax
from jax.experimental import pallas as pl
from jax.experimental.pallas import tpu as pltpu
from jax.experimental.pallas import tpu_sc as plsc

NE = 16
ES = 384
D = 768
T = 2048
TILE = 128
PAD = 4096
GR = PAD // TILE
GS = T // TILE
G = GR + GS
NSLOT = PAD + T


def _cumsum0(a):
    n = a.shape[0]
    sh = 1
    while sh < n:
        z = jnp.zeros((sh,) + a.shape[1:], dtype=a.dtype)
        a = a + jnp.concatenate([z, a[:-sh]], axis=0)
        sh *= 2
    return a


def _cumsum1(a):
    n = a.shape[1]
    sh = 1
    while sh < n:
        z = jnp.zeros(a.shape[:1] + (sh,), dtype=a.dtype)
        a = a + jnp.concatenate([z, a[:, :-sh]], axis=1)
        sh *= 2
    return a


def _router_body(x_ref, wr_ref, br_ref, bias_ref,
                 dest_ref, order_ref, gates_ref, texp_ref):
    xl = x_ref[...]
    logits = jnp.dot(xl, wr_ref[...], preferred_element_type=jnp.float32)
    logits = logits + br_ref[...] + bias_ref[...]
    lmax = jnp.max(logits, axis=1, keepdims=True)
    gate = 1.0 / jnp.sum(jnp.exp(logits - lmax), axis=1, keepdims=True)
    lane = lax.broadcasted_iota(jnp.int32, (T, NE), 1)
    eid = jnp.min(jnp.where(logits == lmax, lane, NE), axis=1, keepdims=True)
    onehot = (lane == eid).astype(jnp.float32)
    counts = jnp.sum(onehot, axis=0, keepdims=True).astype(jnp.int32)
    ptrows = ((counts + TILE - 1) // TILE) * TILE
    poff = _cumsum1(ptrows) - ptrows
    cum = _cumsum0(onehot) - onehot
    rank = jnp.sum(cum * onehot, axis=1, keepdims=True)
    destf = jnp.sum(onehot * poff.astype(jnp.float32), axis=1,
                    keepdims=True) + rank
    dest = destf.astype(jnp.int32)
    dest_ref[...] = dest

    trow = lax.broadcasted_iota(jnp.int32, (G, NE), 0)
    te = jnp.sum((trow * TILE >= poff).astype(jnp.int32), axis=1,
                 keepdims=True) - 1
    gi = lax.broadcasted_iota(jnp.int32, (G, 1), 0)
    texp_ref[...] = jnp.where(gi >= GR, NE, te)

    tok = lax.broadcasted_iota(jnp.int32, (T, 1), 0).astype(jnp.float32)
    for r in range(PAD // 256):
        slots = r * 256 + lax.broadcasted_iota(jnp.int32, (1, 256), 1)
        m = (dest == slots).astype(jnp.float32)
        occ = jnp.sum(m, axis=0, keepdims=True)
        osum = jnp.sum(m * tok, axis=0, keepdims=True).astype(jnp.int32)
        order_ref[r:r + 1, :] = jnp.where(occ > 0.0, osum,
                                          jnp.bitwise_and(slots, T - 1))
        gates_ref[r:r + 1, :] = jnp.sum(m * gate, axis=0, keepdims=True)


def _gelu(h):
    return 0.5 * h * (1.0 + lax.erf(h * 0.7071067811865476))


def _ffn_body(texp_ref, xs_ref, w1_ref, b1_ref, w2_ref, b2_ref,
              g_ref, ys_ref):
    xb = xs_ref[...].astype(jnp.bfloat16)
    h = jnp.dot(xb, w1_ref[0].astype(jnp.bfloat16),
                preferred_element_type=jnp.float32)
    h = _gelu(h + b1_ref[0]).astype(jnp.bfloat16)
    y = jnp.dot(h, w2_ref[0].astype(jnp.bfloat16),
                preferred_element_type=jnp.float32)
    ys_ref[...] = (y + b2_ref[0]) * g_ref[...]


def _shared_body(x_ref, w1_ref, b1_ref, w2_ref, b2_ref, ys_ref):
    xb = x_ref[...].astype(jnp.bfloat16)
    h = jnp.dot(xb, w1_ref[...].astype(jnp.bfloat16),
                preferred_element_type=jnp.float32)
    h = _gelu(h + b1_ref[...]).astype(jnp.bfloat16)
    y = jnp.dot(h, w2_ref[...].astype(jnp.bfloat16),
                preferred_element_type=jnp.float32)
    ys_ref[...] = y + b2_ref[...]


def _add_body(a_ref, b_ref, o_ref):
    o_ref[...] = a_ref[...] + b_ref[...]


def _dispatch_body(x_hbm, order_hbm, xs_hbm, idx_v, rows_v, sem):
    wid = lax.axis_index("s") * 2 + lax.axis_index("c")
    base = wid * (PAD // 32)
    pltpu.sync_copy(order_hbm.at[pl.ds(base, PAD // 32)], idx_v)
    pltpu.async_copy(x_hbm.at[idx_v], rows_v, sem).wait()
    pltpu.sync_copy(rows_v, xs_hbm.at[pl.ds(base, PAD // 32)])


def _combine_body(ys_hbm, dest_hbm, out_hbm, idx_v, rows_v, sem):
    wid = lax.axis_index("s") * 2 + lax.axis_index("c")
    n = T // 32
    base = wid * n
    pltpu.sync_copy(dest_hbm.at[pl.ds(base, n)], idx_v)
    pltpu.async_copy(ys_hbm.at[idx_v], rows_v, sem).wait()
    pltpu.sync_copy(rows_v, out_hbm.at[pl.ds(base, n)])


def _make_router(interpret=False):
    return pl.pallas_call(
        _router_body,
        out_shape=(
            jax.ShapeDtypeStruct((T, 1), jnp.int32),
            jax.ShapeDtypeStruct((PAD // 256, 256), jnp.int32),
            jax.ShapeDtypeStruct((PAD // 256, 256), jnp.float32),
            jax.ShapeDtypeStruct((G, 1), jnp.int32),
        ),
        interpret=interpret,
    )


def _make_ffn(interpret=False):
    grid_spec = pltpu.PrefetchScalarGridSpec(
        num_scalar_prefetch=1,
        grid=(GR,),
        in_specs=[
            pl.BlockSpec((TILE, D), lambda i, te: (i, 0)),
            pl.BlockSpec((1, D, ES), lambda i, te: (te[i], 0, 0)),
            pl.BlockSpec((1, 1, ES), lambda i, te: (te[i], 0, 0)),
            pl.BlockSpec((1, ES, D), lambda i, te: (te[i], 0, 0)),
            pl.BlockSpec((1, 1, D), lambda i, te: (te[i], 0, 0)),
            pl.BlockSpec((TILE, 1), lambda i, te: (i, 0)),
        ],
        out_specs=pl.BlockSpec((TILE, D), lambda i, te: (i, 0)),
    )
    return pl.pallas_call(
        _ffn_body,
        grid_spec=grid_spec,
        out_shape=jax.ShapeDtypeStruct((PAD, D), jnp.float32),
        compiler_params=pltpu.CompilerParams(
            dimension_semantics=("arbitrary",)),
        interpret=interpret,
    )


def _make_shared(interpret=False):
    return pl.pallas_call(
        _shared_body,
        grid=(GS,),
        in_specs=[
            pl.BlockSpec((TILE, D), lambda i: (i, 0)),
            pl.BlockSpec((D, ES), lambda i: (0, 0)),
            pl.BlockSpec((1, ES), lambda i: (0, 0)),
            pl.BlockSpec((ES, D), lambda i: (0, 0)),
            pl.BlockSpec((1, D), lambda i: (0, 0)),
        ],
        out_specs=pl.BlockSpec((TILE, D), lambda i: (i, 0)),
        out_shape=jax.ShapeDtypeStruct((T, D), jnp.float32),
        compiler_params=pltpu.CompilerParams(
            dimension_semantics=("arbitrary",)),
        interpret=interpret,
    )


def _make_add(interpret=False):
    return pl.pallas_call(
        _add_body,
        grid=(8,),
        in_specs=[
            pl.BlockSpec((T // 8, D), lambda i: (i, 0)),
            pl.BlockSpec((T // 8, D), lambda i: (i, 0)),
        ],
        out_specs=pl.BlockSpec((T // 8, D), lambda i: (i, 0)),
        out_shape=jax.ShapeDtypeStruct((T, D), jnp.float32),
        interpret=interpret,
    )


def _make_dispatch():
    mesh = plsc.VectorSubcoreMesh(core_axis_name="c", subcore_axis_name="s")
    return pl.kernel(
        _dispatch_body,
        out_type=jax.ShapeDtypeStruct((PAD, D), jnp.float32),
        mesh=mesh,
        scratch_types=[
            pltpu.VMEM((PAD // 32,), jnp.int32),
            pltpu.VMEM((PAD // 32, D), jnp.float32),
            pltpu.SemaphoreType.DMA,
        ],
    )


def _make_combine():
    mesh = plsc.VectorSubcoreMesh(core_axis_name="c", subcore_axis_name="s")
    return pl.kernel(
        _combine_body,
        out_type=jax.ShapeDtypeStruct((T, D), jnp.float32),
        mesh=mesh,
        scratch_types=[
            pltpu.VMEM((T // 32,), jnp.int32),
            pltpu.VMEM((T // 32, D), jnp.float32),
            pltpu.SemaphoreType.DMA,
        ],
    )


@jax.jit
def kernel(x, Wr, br, bias, W1, b1, W2, b2):
    x2 = x.reshape(T, D)
    dest2d, order2d, gates2d, texp2d = _make_router()(
        x2, Wr, br.reshape(1, NE), bias.reshape(1, NE))
    order = order2d.reshape(PAD)
    dest = dest2d.reshape(T)
    gates = gates2d.reshape(PAD, 1)
    texp = texp2d.reshape(G)
    xs = _make_dispatch()(x2, order)
    ys_sh = _make_shared()(x2, W1[NE], b1[NE].reshape(1, ES),
                           W2[NE], b2[NE].reshape(1, D))
    ys = _make_ffn()(texp, xs, W1, b1.reshape(NE + 1, 1, ES), W2,
                     b2.reshape(NE + 1, 1, D), gates)
    ys_r = _make_combine()(ys, dest)
    out = _make_add()(ys_r, ys_sh)
    return out.reshape(x.shape)


# speedup vs baseline: 1.3143x; 1.1427x over previous
import functools

import jax
import jax.numpy as jnp
from jax import lax
from jax.experimental import pallas as pl
from jax.experimental.pallas import tpu as pltpu
from jax.experimental.pallas import tpu_sc as plsc

NE = 16
ES = 384
D = 768
T = 2048
TILE = 128
PAD = 4096
GR = PAD // TILE
GS = T // TILE
G = GR + GS
NSLOT = PAD + T


def _cumsum0(a):
    n = a.shape[0]
    sh = 1
    while sh < n:
        z = jnp.zeros((sh,) + a.shape[1:], dtype=a.dtype)
        a = a + jnp.concatenate([z, a[:-sh]], axis=0)
        sh *= 2
    return a


def _cumsum1(a):
    n = a.shape[1]
    sh = 1
    while sh < n:
        z = jnp.zeros(a.shape[:1] + (sh,), dtype=a.dtype)
        a = a + jnp.concatenate([z, a[:, :-sh]], axis=1)
        sh *= 2
    return a


def _router_body(x_ref, wr_ref, br_ref, bias_ref,
                 dest_ref, order_ref, gates_ref, texp_ref, nact_ref):
    xl = x_ref[...]
    logits = jnp.dot(xl, wr_ref[...], preferred_element_type=jnp.float32)
    logits = logits + br_ref[...] + bias_ref[...]
    lmax = jnp.max(logits, axis=1, keepdims=True)
    gate = 1.0 / jnp.sum(jnp.exp(logits - lmax), axis=1, keepdims=True)
    lane = lax.broadcasted_iota(jnp.int32, (T, NE), 1)
    eid = jnp.min(jnp.where(logits == lmax, lane, NE), axis=1, keepdims=True)
    onehot = (lane == eid).astype(jnp.float32)
    counts = jnp.sum(onehot, axis=0, keepdims=True).astype(jnp.int32)
    ptrows = ((counts + TILE - 1) // TILE) * TILE
    poff = _cumsum1(ptrows) - ptrows
    cum = _cumsum0(onehot) - onehot
    rank = jnp.sum(cum * onehot, axis=1, keepdims=True)
    destf = jnp.sum(onehot * poff.astype(jnp.float32), axis=1,
                    keepdims=True) + rank
    dest = destf.astype(jnp.int32)
    dest_ref[...] = dest

    trow = lax.broadcasted_iota(jnp.int32, (G, NE), 0)
    te = jnp.sum((trow * TILE >= poff).astype(jnp.int32), axis=1,
                 keepdims=True) - 1
    gi = lax.broadcasted_iota(jnp.int32, (G, 1), 0)
    texp_ref[...] = jnp.where(gi >= GR, NE, te)
    nact_ref[...] = jnp.sum(ptrows, axis=1, keepdims=True) // TILE

    tok = lax.broadcasted_iota(jnp.int32, (T, 1), 0).astype(jnp.float32)
    tg = tok + gate
    for r in range(PAD // 256):
        slots = r * 256 + lax.broadcasted_iota(jnp.int32, (1, 256), 1)
        m = (dest == slots).astype(jnp.float32)
        v = jnp.sum(m * tg, axis=0, keepdims=True)
        o = jnp.floor(v)
        order_ref[r:r + 1, :] = jnp.where(v > 0.0, o.astype(jnp.int32),
                                          jnp.bitwise_and(slots, T - 1))
        gates_ref[r:r + 1, :] = v - o


def _gelu(h):
    return 0.5 * h * (1.0 + lax.erf(h * 0.7071067811865476))


def _ffn_body(texp_ref, nact_ref, xs_ref, w1_ref, b1_ref, w2_ref, b2_ref,
              g_ref, ys_ref):
    @pl.when(pl.program_id(0) < nact_ref[0])
    def _():
        xb = xs_ref[...].astype(jnp.bfloat16)
        h = jnp.dot(xb, w1_ref[0].astype(jnp.bfloat16),
                    preferred_element_type=jnp.float32)
        h = _gelu(h + b1_ref[0]).astype(jnp.bfloat16)
        y = jnp.dot(h, w2_ref[0].astype(jnp.bfloat16),
                    preferred_element_type=jnp.float32)
        ys_ref[...] = (y + b2_ref[0]) * g_ref[...]


def _shared_body(x_ref, w1_ref, b1_ref, w2_ref, b2_ref, ys_ref):
    xb = x_ref[...].astype(jnp.bfloat16)
    h = jnp.dot(xb, w1_ref[0].astype(jnp.bfloat16),
                preferred_element_type=jnp.float32)
    h = _gelu(h + b1_ref[0]).astype(jnp.bfloat16)
    y = jnp.dot(h, w2_ref[0].astype(jnp.bfloat16),
                preferred_element_type=jnp.float32)
    ys_ref[...] = y + b2_ref[0]


def _add_body(a_ref, b_ref, o_ref):
    o_ref[...] = a_ref[...] + b_ref[...]


def _dispatch_body(x_hbm, order_hbm, xs_hbm, idx_v, rows_v, sem):
    wid = lax.axis_index("s") * 2 + lax.axis_index("c")
    base = wid * (PAD // 32)
    pltpu.sync_copy(order_hbm.at[pl.ds(base, PAD // 32)], idx_v)
    pltpu.async_copy(x_hbm.at[idx_v], rows_v, sem).wait()
    pltpu.sync_copy(rows_v, xs_hbm.at[pl.ds(base, PAD // 32)])


def _combine_body(ys_hbm, dest_hbm, out_hbm, idx_v, rows_v, sem):
    wid = lax.axis_index("s") * 2 + lax.axis_index("c")
    n = T // 32
    base = wid * n
    pltpu.sync_copy(dest_hbm.at[pl.ds(base, n)], idx_v)
    pltpu.async_copy(ys_hbm.at[idx_v], rows_v, sem).wait()
    pltpu.sync_copy(rows_v, out_hbm.at[pl.ds(base, n)])


def _make_router(interpret=False):
    return pl.pallas_call(
        _router_body,
        out_shape=(
            jax.ShapeDtypeStruct((T, 1), jnp.int32),
            jax.ShapeDtypeStruct((PAD // 256, 256), jnp.int32),
            jax.ShapeDtypeStruct((PAD // 256, 256), jnp.float32),
            jax.ShapeDtypeStruct((G, 1), jnp.int32),
            jax.ShapeDtypeStruct((1, 1), jnp.int32),
        ),
        interpret=interpret,
    )


def _make_ffn(interpret=False):
    def _act(i, na):
        return jnp.where(i < na[0], i, na[0] - 1)

    def _texp(i, te, na):
        return te[jnp.where(i < na[0], i, na[0] - 1)]

    grid_spec = pltpu.PrefetchScalarGridSpec(
        num_scalar_prefetch=2,
        grid=(GR,),
        in_specs=[
            pl.BlockSpec((TILE, D), lambda i, te, na: (_act(i, na), 0)),
            pl.BlockSpec((1, D, ES), lambda i, te, na: (_texp(i, te, na), 0, 0)),
            pl.BlockSpec((1, 1, ES), lambda i, te, na: (_texp(i, te, na), 0, 0)),
            pl.BlockSpec((1, ES, D), lambda i, te, na: (_texp(i, te, na), 0, 0)),
            pl.BlockSpec((1, 1, D), lambda i, te, na: (_texp(i, te, na), 0, 0)),
            pl.BlockSpec((TILE, 1), lambda i, te, na: (_act(i, na), 0)),
        ],
        out_specs=pl.BlockSpec((TILE, D), lambda i, te, na: (_act(i, na), 0)),
    )
    return pl.pallas_call(
        _ffn_body,
        grid_spec=grid_spec,
        out_shape=jax.ShapeDtypeStruct((PAD, D), jnp.float32),
        compiler_params=pltpu.CompilerParams(
            dimension_semantics=("arbitrary",)),
        interpret=interpret,
    )


def _make_shared(interpret=False):
    return pl.pallas_call(
        _shared_body,
        grid=(GS,),
        in_specs=[
            pl.BlockSpec((TILE, D), lambda i: (i, 0)),
            pl.BlockSpec((1, D, ES), lambda i: (NE, 0, 0)),
            pl.BlockSpec((1, 1, ES), lambda i: (NE, 0, 0)),
            pl.BlockSpec((1, ES, D), lambda i: (NE, 0, 0)),
            pl.BlockSpec((1, 1, D), lambda i: (NE, 0, 0)),
        ],
        out_specs=pl.BlockSpec((TILE, D), lambda i: (i, 0)),
        out_shape=jax.ShapeDtypeStruct((T, D), jnp.float32),
        compiler_params=pltpu.CompilerParams(
            dimension_semantics=("arbitrary",)),
        interpret=interpret,
    )


def _make_add(interpret=False):
    return pl.pallas_call(
        _add_body,
        grid=(8,),
        in_specs=[
            pl.BlockSpec((T // 8, D), lambda i: (i, 0)),
            pl.BlockSpec((T // 8, D), lambda i: (i, 0)),
        ],
        out_specs=pl.BlockSpec((T // 8, D), lambda i: (i, 0)),
        out_shape=jax.ShapeDtypeStruct((T, D), jnp.float32),
        interpret=interpret,
    )


def _make_dispatch():
    mesh = plsc.VectorSubcoreMesh(core_axis_name="c", subcore_axis_name="s")
    return pl.kernel(
        _dispatch_body,
        out_type=jax.ShapeDtypeStruct((PAD, D), jnp.float32),
        mesh=mesh,
        scratch_types=[
            pltpu.VMEM((PAD // 32,), jnp.int32),
            pltpu.VMEM((PAD // 32, D), jnp.float32),
            pltpu.SemaphoreType.DMA,
        ],
    )


def _make_combine():
    mesh = plsc.VectorSubcoreMesh(core_axis_name="c", subcore_axis_name="s")
    return pl.kernel(
        _combine_body,
        out_type=jax.ShapeDtypeStruct((T, D), jnp.float32),
        mesh=mesh,
        scratch_types=[
            pltpu.VMEM((T // 32,), jnp.int32),
            pltpu.VMEM((T // 32, D), jnp.float32),
            pltpu.SemaphoreType.DMA,
        ],
    )


@jax.jit
def kernel(x, Wr, br, bias, W1, b1, W2, b2):
    x2 = x.reshape(T, D)
    dest2d, order2d, gates2d, texp2d, nact2d = _make_router()(
        x2, Wr, br.reshape(1, NE), bias.reshape(1, NE))
    order = order2d.reshape(PAD)
    dest = dest2d.reshape(T)
    gates = gates2d.reshape(PAD, 1)
    texp = texp2d.reshape(G)
    nact = nact2d.reshape(1)
    b1r = b1.reshape(NE + 1, 1, ES)
    b2r = b2.reshape(NE + 1, 1, D)
    xs = _make_dispatch()(x2, order)
    ys_sh = _make_shared()(x2, W1, b1r, W2, b2r)
    ys = _make_ffn()(texp, nact, xs, W1, b1r, W2, b2r, gates)
    ys_r = _make_combine()(ys, dest)
    out = _make_add()(ys_r, ys_sh)
    return out.reshape(x.shape)


# speedup vs baseline: 1.3614x; 1.0359x over previous
import functools

import jax
import jax.numpy as jnp
from jax import lax
from jax.experimental import pallas as pl
from jax.experimental.pallas import tpu as pltpu
from jax.experimental.pallas import tpu_sc as plsc

NE = 16
ES = 384
D = 768
T = 2048
TILE = 128
PAD = 4096
GR = PAD // TILE
GS = T // TILE
G = GR + GS
NSLOT = PAD + T


def _cumsum0(a):
    n = a.shape[0]
    sh = 1
    while sh < n:
        z = jnp.zeros((sh,) + a.shape[1:], dtype=a.dtype)
        a = a + jnp.concatenate([z, a[:-sh]], axis=0)
        sh *= 2
    return a


def _cumsum1(a):
    n = a.shape[1]
    sh = 1
    while sh < n:
        z = jnp.zeros(a.shape[:1] + (sh,), dtype=a.dtype)
        a = a + jnp.concatenate([z, a[:, :-sh]], axis=1)
        sh *= 2
    return a


def _router_body(x_ref, wr_ref, br_ref, bias_ref,
                 dest_ref, order_ref, gates_ref, texp_ref, nact_ref):
    xl = x_ref[...]
    logits = jnp.dot(xl, wr_ref[...], preferred_element_type=jnp.float32)
    logits = logits + br_ref[...] + bias_ref[...]
    lmax = jnp.max(logits, axis=1, keepdims=True)
    gate = 1.0 / jnp.sum(jnp.exp(logits - lmax), axis=1, keepdims=True)
    lane = lax.broadcasted_iota(jnp.int32, (T, NE), 1)
    eid = jnp.min(jnp.where(logits == lmax, lane, NE), axis=1, keepdims=True)
    onehot = (lane == eid).astype(jnp.float32)
    counts = jnp.sum(onehot, axis=0, keepdims=True).astype(jnp.int32)
    ptrows = ((counts + TILE - 1) // TILE) * TILE
    poff = _cumsum1(ptrows) - ptrows
    cum = _cumsum0(onehot) - onehot
    rank = jnp.sum(cum * onehot, axis=1, keepdims=True)
    destf = jnp.sum(onehot * poff.astype(jnp.float32), axis=1,
                    keepdims=True) + rank
    dest = destf.astype(jnp.int32)
    dest_ref[...] = dest

    trow = lax.broadcasted_iota(jnp.int32, (G, NE), 0)
    te = jnp.sum((trow * TILE >= poff).astype(jnp.int32), axis=1,
                 keepdims=True) - 1
    gi = lax.broadcasted_iota(jnp.int32, (G, 1), 0)
    texp_ref[...] = jnp.where(gi >= GR, NE, te)
    nact_ref[...] = jnp.sum(ptrows, axis=1, keepdims=True) // TILE

    tok = lax.broadcasted_iota(jnp.int32, (T, 1), 0).astype(jnp.float32)
    tg = tok + gate
    for r in range(PAD // 256):
        slots = r * 256 + lax.broadcasted_iota(jnp.int32, (1, 256), 1)
        m = (dest == slots).astype(jnp.float32)
        v = jnp.sum(m * tg, axis=0, keepdims=True)
        o = jnp.floor(v)
        order_ref[r:r + 1, :] = jnp.where(v > 0.0, o.astype(jnp.int32),
                                          jnp.bitwise_and(slots, T - 1))
        gates_ref[r:r + 1, :] = v - o


def _gelu(h):
    return 0.5 * h * (1.0 + lax.erf(h * 0.7071067811865476))


def _ffn_body(texp_ref, nact_ref, xs_ref, w1_ref, b1_ref, w2_ref, b2_ref,
              g_ref, ys_ref):
    @pl.when(pl.program_id(0) < nact_ref[0])
    def _():
        xb = xs_ref[...].astype(jnp.bfloat16)
        h = jnp.dot(xb, w1_ref[0].astype(jnp.bfloat16),
                    preferred_element_type=jnp.float32)
        h = _gelu(h + b1_ref[0]).astype(jnp.bfloat16)
        y = jnp.dot(h, w2_ref[0].astype(jnp.bfloat16),
                    preferred_element_type=jnp.float32)
        ys_ref[...] = (y + b2_ref[0]) * g_ref[...]


def _shared_body(x_ref, w1_ref, b1_ref, w2_ref, b2_ref, yr_ref, out_ref):
    xb = x_ref[...].astype(jnp.bfloat16)
    h = jnp.dot(xb, w1_ref[0].astype(jnp.bfloat16),
                preferred_element_type=jnp.float32)
    h = _gelu(h + b1_ref[0]).astype(jnp.bfloat16)
    y = jnp.dot(h, w2_ref[0].astype(jnp.bfloat16),
                preferred_element_type=jnp.float32)
    out_ref[...] = y + b2_ref[0] + yr_ref[...]


def _dispatch_body(x_hbm, order_hbm, xs_hbm, idx_v, rows_v, sem):
    wid = lax.axis_index("s") * 2 + lax.axis_index("c")
    base = wid * (PAD // 32)
    pltpu.sync_copy(order_hbm.at[pl.ds(base, PAD // 32)], idx_v)
    pltpu.async_copy(x_hbm.at[idx_v], rows_v, sem).wait()
    pltpu.sync_copy(rows_v, xs_hbm.at[pl.ds(base, PAD // 32)])


def _combine_body(ys_hbm, dest_hbm, out_hbm, idx_v, rows_v, sem):
    wid = lax.axis_index("s") * 2 + lax.axis_index("c")
    n = T // 32
    base = wid * n
    pltpu.sync_copy(dest_hbm.at[pl.ds(base, n)], idx_v)
    pltpu.async_copy(ys_hbm.at[idx_v], rows_v, sem).wait()
    pltpu.sync_copy(rows_v, out_hbm.at[pl.ds(base, n)])


def _make_router(interpret=False):
    return pl.pallas_call(
        _router_body,
        out_shape=(
            jax.ShapeDtypeStruct((T, 1), jnp.int32),
            jax.ShapeDtypeStruct((PAD // 256, 256), jnp.int32),
            jax.ShapeDtypeStruct((PAD // 256, 256), jnp.float32),
            jax.ShapeDtypeStruct((G, 1), jnp.int32),
            jax.ShapeDtypeStruct((1, 1), jnp.int32),
        ),
        interpret=interpret,
    )


def _make_ffn(interpret=False):
    def _act(i, na):
        return jnp.where(i < na[0], i, na[0] - 1)

    def _texp(i, te, na):
        return te[jnp.where(i < na[0], i, na[0] - 1)]

    grid_spec = pltpu.PrefetchScalarGridSpec(
        num_scalar_prefetch=2,
        grid=(GR,),
        in_specs=[
            pl.BlockSpec((TILE, D), lambda i, te, na: (_act(i, na), 0)),
            pl.BlockSpec((1, D, ES), lambda i, te, na: (_texp(i, te, na), 0, 0)),
            pl.BlockSpec((1, 1, ES), lambda i, te, na: (_texp(i, te, na), 0, 0)),
            pl.BlockSpec((1, ES, D), lambda i, te, na: (_texp(i, te, na), 0, 0)),
            pl.BlockSpec((1, 1, D), lambda i, te, na: (_texp(i, te, na), 0, 0)),
            pl.BlockSpec((TILE, 1), lambda i, te, na: (_act(i, na), 0)),
        ],
        out_specs=pl.BlockSpec((TILE, D), lambda i, te, na: (_act(i, na), 0)),
    )
    return pl.pallas_call(
        _ffn_body,
        grid_spec=grid_spec,
        out_shape=jax.ShapeDtypeStruct((PAD, D), jnp.float32),
        compiler_params=pltpu.CompilerParams(
            dimension_semantics=("arbitrary",)),
        interpret=interpret,
    )


def _make_shared(interpret=False):
    return pl.pallas_call(
        _shared_body,
        grid=(GS,),
        in_specs=[
            pl.BlockSpec((TILE, D), lambda i: (i, 0)),
            pl.BlockSpec((1, D, ES), lambda i: (NE, 0, 0)),
            pl.BlockSpec((1, 1, ES), lambda i: (NE, 0, 0)),
            pl.BlockSpec((1, ES, D), lambda i: (NE, 0, 0)),
            pl.BlockSpec((1, 1, D), lambda i: (NE, 0, 0)),
            pl.BlockSpec((TILE, D), lambda i: (i, 0)),
        ],
        out_specs=pl.BlockSpec((TILE, D), lambda i: (i, 0)),
        out_shape=jax.ShapeDtypeStruct((T, D), jnp.float32),
        compiler_params=pltpu.CompilerParams(
            dimension_semantics=("arbitrary",)),
        interpret=interpret,
    )


def _make_dispatch():
    mesh = plsc.VectorSubcoreMesh(core_axis_name="c", subcore_axis_name="s")
    return pl.kernel(
        _dispatch_body,
        out_type=jax.ShapeDtypeStruct((PAD, D), jnp.float32),
        mesh=mesh,
        scratch_types=[
            pltpu.VMEM((PAD // 32,), jnp.int32),
            pltpu.VMEM((PAD // 32, D), jnp.float32),
            pltpu.SemaphoreType.DMA,
        ],
    )


def _make_combine():
    mesh = plsc.VectorSubcoreMesh(core_axis_name="c", subcore_axis_name="s")
    return pl.kernel(
        _combine_body,
        out_type=jax.ShapeDtypeStruct((T, D), jnp.float32),
        mesh=mesh,
        scratch_types=[
            pltpu.VMEM((T // 32,), jnp.int32),
            pltpu.VMEM((T // 32, D), jnp.float32),
            pltpu.SemaphoreType.DMA,
        ],
    )


@jax.jit
def kernel(x, Wr, br, bias, W1, b1, W2, b2):
    x2 = x.reshape(T, D)
    dest2d, order2d, gates2d, texp2d, nact2d = _make_router()(
        x2, Wr, br.reshape(1, NE), bias.reshape(1, NE))
    order = order2d.reshape(PAD)
    dest = dest2d.reshape(T)
    gates = gates2d.reshape(PAD, 1)
    texp = texp2d.reshape(G)
    nact = nact2d.reshape(1)
    b1r = b1.reshape(NE + 1, 1, ES)
    b2r = b2.reshape(NE + 1, 1, D)
    xs = _make_dispatch()(x2, order)
    ys = _make_ffn()(texp, nact, xs, W1, b1r, W2, b2r, gates)
    ys_r = _make_combine()(ys, dest)
    out = _make_shared()(x2, W1, b1r, W2, b2r, ys_r)
    return out.reshape(x.shape)


# speedup vs baseline: 1.3622x; 1.0005x over previous
import functools

import jax
import jax.numpy as jnp
from jax import lax
from jax.experimental import pallas as pl
from jax.experimental.pallas import tpu as pltpu
from jax.experimental.pallas import tpu_sc as plsc

NE = 16
ES = 384
D = 768
T = 2048
TILE = 128
PAD = 4096
GR = PAD // TILE
GS = T // TILE
G = GR + GS
NSLOT = PAD + T


def _cumsum0(a):
    n = a.shape[0]
    sh = 1
    while sh < n:
        z = jnp.zeros((sh,) + a.shape[1:], dtype=a.dtype)
        a = a + jnp.concatenate([z, a[:-sh]], axis=0)
        sh *= 2
    return a


def _cumsum1(a):
    n = a.shape[1]
    sh = 1
    while sh < n:
        z = jnp.zeros(a.shape[:1] + (sh,), dtype=a.dtype)
        a = a + jnp.concatenate([z, a[:, :-sh]], axis=1)
        sh *= 2
    return a


def _router_body(x_ref, wr_ref, br_ref, bias_ref,
                 dest_ref, order_ref, gates_ref, texp_ref, nact_ref):
    xl = x_ref[...]
    logits = jnp.dot(xl, wr_ref[...], preferred_element_type=jnp.float32)
    logits = logits + br_ref[...] + bias_ref[...]
    lmax = jnp.max(logits, axis=1, keepdims=True)
    gate = 1.0 / jnp.sum(jnp.exp(logits - lmax), axis=1, keepdims=True)
    lane = lax.broadcasted_iota(jnp.int32, (T, NE), 1)
    eid = jnp.min(jnp.where(logits == lmax, lane, NE), axis=1, keepdims=True)
    onehot = (lane == eid).astype(jnp.float32)
    counts = jnp.sum(onehot, axis=0, keepdims=True).astype(jnp.int32)
    ptrows = ((counts + TILE - 1) // TILE) * TILE
    poff = _cumsum1(ptrows) - ptrows
    cum = _cumsum0(onehot) - onehot
    rank = jnp.sum(cum * onehot, axis=1, keepdims=True)
    destf = jnp.sum(onehot * poff.astype(jnp.float32), axis=1,
                    keepdims=True) + rank
    dest = destf.astype(jnp.int32)
    dest_ref[...] = dest

    trow = lax.broadcasted_iota(jnp.int32, (G, NE), 0)
    te = jnp.sum((trow * TILE >= poff).astype(jnp.int32), axis=1,
                 keepdims=True) - 1
    gi = lax.broadcasted_iota(jnp.int32, (G, 1), 0)
    texp_ref[...] = jnp.where(gi >= GR, NE, te)
    nact_ref[...] = jnp.sum(ptrows, axis=1, keepdims=True) // TILE

    tok = lax.broadcasted_iota(jnp.int32, (T, 1), 0).astype(jnp.float32)
    tg = tok + gate
    for r in range(PAD // 256):
        slots = r * 256 + lax.broadcasted_iota(jnp.int32, (1, 256), 1)
        m = (dest == slots).astype(jnp.float32)
        v = jnp.sum(m * tg, axis=0, keepdims=True)
        o = jnp.floor(v)
        order_ref[r:r + 1, :] = jnp.where(v > 0.0, o.astype(jnp.int32),
                                          jnp.bitwise_and(slots, T - 1))
        gates_ref[r:r + 1, :] = v - o


def _gelu(h):
    return 0.5 * h * (1.0 + lax.erf(h * 0.7071067811865476))


def _ffn_body(texp_ref, nact_ref, xs_ref, w1_ref, b1_ref, w2_ref, b2_ref,
              g_ref, ys_ref):
    @pl.when(pl.program_id(0) < nact_ref[0])
    def _():
        h = jnp.dot(xs_ref[...], w1_ref[0],
                    preferred_element_type=jnp.float32)
        h = _gelu(h + b1_ref[0])
        y = jnp.dot(h, w2_ref[0], preferred_element_type=jnp.float32)
        ys_ref[...] = (y + b2_ref[0]) * g_ref[...]


def _shared_body(x_ref, w1_ref, b1_ref, w2_ref, b2_ref, yr_ref, out_ref):
    h = jnp.dot(x_ref[...], w1_ref[0], preferred_element_type=jnp.float32)
    h = _gelu(h + b1_ref[0])
    y = jnp.dot(h, w2_ref[0], preferred_element_type=jnp.float32)
    out_ref[...] = y + b2_ref[0] + yr_ref[...]


def _dispatch_body(x_hbm, order_hbm, xs_hbm, idx_v, rows_v, sem):
    wid = lax.axis_index("s") * 2 + lax.axis_index("c")
    base = wid * (PAD // 32)
    pltpu.sync_copy(order_hbm.at[pl.ds(base, PAD // 32)], idx_v)
    pltpu.async_copy(x_hbm.at[idx_v], rows_v, sem).wait()
    pltpu.sync_copy(rows_v, xs_hbm.at[pl.ds(base, PAD // 32)])


def _combine_body(ys_hbm, dest_hbm, out_hbm, idx_v, rows_v, sem):
    wid = lax.axis_index("s") * 2 + lax.axis_index("c")
    n = T // 32
    base = wid * n
    pltpu.sync_copy(dest_hbm.at[pl.ds(base, n)], idx_v)
    pltpu.async_copy(ys_hbm.at[idx_v], rows_v, sem).wait()
    pltpu.sync_copy(rows_v, out_hbm.at[pl.ds(base, n)])


def _make_router(interpret=False):
    return pl.pallas_call(
        _router_body,
        out_shape=(
            jax.ShapeDtypeStruct((T, 1), jnp.int32),
            jax.ShapeDtypeStruct((PAD // 256, 256), jnp.int32),
            jax.ShapeDtypeStruct((PAD // 256, 256), jnp.float32),
            jax.ShapeDtypeStruct((G, 1), jnp.int32),
            jax.ShapeDtypeStruct((1, 1), jnp.int32),
        ),
        interpret=interpret,
    )


def _make_ffn(interpret=False):
    def _act(i, na):
        return jnp.where(i < na[0], i, na[0] - 1)

    def _texp(i, te, na):
        return te[jnp.where(i < na[0], i, na[0] - 1)]

    grid_spec = pltpu.PrefetchScalarGridSpec(
        num_scalar_prefetch=2,
        grid=(GR,),
        in_specs=[
            pl.BlockSpec((TILE, D), lambda i, te, na: (_act(i, na), 0)),
            pl.BlockSpec((1, D, ES), lambda i, te, na: (_texp(i, te, na), 0, 0)),
            pl.BlockSpec((1, 1, ES), lambda i, te, na: (_texp(i, te, na), 0, 0)),
            pl.BlockSpec((1, ES, D), lambda i, te, na: (_texp(i, te, na), 0, 0)),
            pl.BlockSpec((1, 1, D), lambda i, te, na: (_texp(i, te, na), 0, 0)),
            pl.BlockSpec((TILE, 1), lambda i, te, na: (_act(i, na), 0)),
        ],
        out_specs=pl.BlockSpec((TILE, D), lambda i, te, na: (_act(i, na), 0)),
    )
    return pl.pallas_call(
        _ffn_body,
        grid_spec=grid_spec,
        out_shape=jax.ShapeDtypeStruct((PAD, D), jnp.float32),
        compiler_params=pltpu.CompilerParams(
            dimension_semantics=("arbitrary",)),
        interpret=interpret,
    )


def _make_shared(interpret=False):
    return pl.pallas_call(
        _shared_body,
        grid=(GS,),
        in_specs=[
            pl.BlockSpec((TILE, D), lambda i: (i, 0)),
            pl.BlockSpec((1, D, ES), lambda i: (NE, 0, 0)),
            pl.BlockSpec((1, 1, ES), lambda i: (NE, 0, 0)),
            pl.BlockSpec((1, ES, D), lambda i: (NE, 0, 0)),
            pl.BlockSpec((1, 1, D), lambda i: (NE, 0, 0)),
            pl.BlockSpec((TILE, D), lambda i: (i, 0)),
        ],
        out_specs=pl.BlockSpec((TILE, D), lambda i: (i, 0)),
        out_shape=jax.ShapeDtypeStruct((T, D), jnp.float32),
        compiler_params=pltpu.CompilerParams(
            dimension_semantics=("arbitrary",)),
        interpret=interpret,
    )


def _make_dispatch():
    mesh = plsc.VectorSubcoreMesh(core_axis_name="c", subcore_axis_name="s")
    return pl.kernel(
        _dispatch_body,
        out_type=jax.ShapeDtypeStruct((PAD, D), jnp.float32),
        mesh=mesh,
        scratch_types=[
            pltpu.VMEM((PAD // 32,), jnp.int32),
            pltpu.VMEM((PAD // 32, D), jnp.float32),
            pltpu.SemaphoreType.DMA,
        ],
    )


def _make_combine():
    mesh = plsc.VectorSubcoreMesh(core_axis_name="c", subcore_axis_name="s")
    return pl.kernel(
        _combine_body,
        out_type=jax.ShapeDtypeStruct((T, D), jnp.float32),
        mesh=mesh,
        scratch_types=[
            pltpu.VMEM((T // 32,), jnp.int32),
            pltpu.VMEM((T // 32, D), jnp.float32),
            pltpu.SemaphoreType.DMA,
        ],
    )


@jax.jit
def kernel(x, Wr, br, bias, W1, b1, W2, b2):
    x2 = x.reshape(T, D)
    dest2d, order2d, gates2d, texp2d, nact2d = _make_router()(
        x2, Wr, br.reshape(1, NE), bias.reshape(1, NE))
    order = order2d.reshape(PAD)
    dest = dest2d.reshape(T)
    gates = gates2d.reshape(PAD, 1)
    texp = texp2d.reshape(G)
    nact = nact2d.reshape(1)
    b1r = b1.reshape(NE + 1, 1, ES)
    b2r = b2.reshape(NE + 1, 1, D)
    xs = _make_dispatch()(x2, order)
    ys = _make_ffn()(texp, nact, xs, W1, b1r, W2, b2r, gates)
    ys_r = _make_combine()(ys, dest)
    out = _make_shared()(x2, W1, b1r, W2, b2r, ys_r)
    return out.reshape(x.shape)
